# baseline (speedup 1.0000x reference)
import functools

import jax
import jax.numpy as jnp
from jax import lax
from jax.experimental import pallas as pl
from jax.experimental.pallas import tpu as pltpu
from jax.experimental.pallas import tpu_sc as plsc

_NUM_NODES = 1000000
_D = 16
_B = 16384
_NS = 16
_PER_TILE = _B // _NS
_CH = 128
_NCH = _PER_TILE // _CH
_ROWS_PER_TILE = _PER_TILE // _CH

_mesh = plsc.VectorSubcoreMesh(core_axis_name="c", subcore_axis_name="s")


@functools.partial(
    pl.kernel,
    out_type=jax.ShapeDtypeStruct((_D, _B), jnp.float32),
    mesh=_mesh,
    scratch_types=[
        pltpu.VMEM((_NCH // 2, _CH), jnp.int32),
        pltpu.VMEM((_D, _PER_TILE // 2), jnp.float32),
        pltpu.SemaphoreType.DMA,
    ],
    compiler_params=pltpu.CompilerParams(use_tc_tiling_on_sc=False),
)
def _sc_gather(wt, node2d, embt, idx_v, val_v, sem):
    c = lax.axis_index("c")
    s = lax.axis_index("s")
    w = s * 2 + c
    per_w = _PER_TILE // 2
    nch = _NCH // 2
    base = w * per_w
    rowbase = w * (per_w // _CH)

    pltpu.sync_copy(node2d.at[pl.ds(rowbase, nch)], idx_v)
    for j in range(nch):
        descs = [
            pltpu.async_copy(
                wt.at[d].at[idx_v.at[j]],
                val_v.at[d, pl.ds(j * _CH, _CH)],
                sem,
            )
            for d in range(_D)
        ]
        for desc in descs:
            desc.wait()
    for d in range(_D):
        pltpu.sync_copy(val_v.at[d], embt.at[d, pl.ds(base, per_w)])


@functools.partial(
    pl.kernel,
    out_type=(),
    mesh=_mesh,
    scratch_types=[
        pltpu.VMEM_SHARED((_NUM_NODES,), jnp.int32),
        pltpu.VMEM_SHARED((_D, _B), jnp.float32),
        pltpu.VMEM_SHARED((_B,), jnp.float32),
        pltpu.VMEM((_NCH, _CH), jnp.int32),
        pltpu.VMEM((_NCH, _CH), jnp.int32),
        pltpu.VMEM((_NCH, _CH), jnp.int32),
        pltpu.VMEM((_D, _PER_TILE), jnp.float32),
        pltpu.VMEM((_D, _PER_TILE), jnp.float32),
        pltpu.VMEM((_PER_TILE,), jnp.float32),
        pltpu.VMEM((_PER_TILE,), jnp.float32),
        pltpu.VMEM((_CH,), jnp.float32),
        pltpu.SemaphoreType.DMA,
    ],
    compiler_params=pltpu.CompilerParams(use_tc_tiling_on_sc=False),
)
def _sc_update(node2d, updt, outw, slot_tab, acc, cnt, idx_v, ids_v, rep_v,
               upd_v, val_v, cnt_v, zrow_v, oc_v, sem):
    c = lax.axis_index("c")
    s = lax.axis_index("s")
    base = s * _PER_TILE
    rowbase = s * _ROWS_PER_TILE

    @pl.when(c == 0)
    def _():
        pltpu.sync_copy(node2d.at[pl.ds(rowbase, _NCH)], idx_v)
        for j in range(_NCH):
            for g in range(_CH // _D):
                ids_v[j, pl.ds(g * _D, _D)] = (
                    base + j * _CH + g * _D + lax.iota(jnp.int32, _D)
                )
        for g in range(_PER_TILE // _D):
            zrow_v[pl.ds(g * _D, _D)] = jnp.zeros((_D,), jnp.float32)
        for g in range(_CH // _D):
            oc_v[pl.ds(g * _D, _D)] = jnp.ones((_D,), jnp.float32)
        for d in range(_D):
            pltpu.sync_copy(updt.at[d, pl.ds(base, _PER_TILE)], upd_v.at[d])
        for d in range(_D):
            pltpu.sync_copy(zrow_v, acc.at[d, pl.ds(base, _PER_TILE)])
        pltpu.sync_copy(zrow_v, cnt.at[pl.ds(base, _PER_TILE)])
        for j in range(_NCH):
            pltpu.sync_copy(ids_v.at[j], slot_tab.at[idx_v.at[j]])

    plsc.subcore_barrier()

    @pl.when(c == 0)
    def _():
        for j in range(_NCH):
            pltpu.sync_copy(slot_tab.at[idx_v.at[j]], rep_v.at[j])
        for j in range(_NCH):
            descs = [
                pltpu.async_copy(
                    upd_v.at[d, pl.ds(j * _CH, _CH)],
                    acc.at[d].at[rep_v.at[j]],
                    sem,
                    add=True,
                )
                for d in range(_D)
            ]
            descs.append(
                pltpu.async_copy(oc_v, cnt.at[rep_v.at[j]], sem, add=True)
            )
            for desc in descs:
                desc.wait()

    plsc.subcore_barrier()

    @pl.when(c == 0)
    def _():
        for j in range(_NCH):
            descs = [
                pltpu.async_copy(
                    acc.at[d].at[rep_v.at[j]],
                    val_v.at[d, pl.ds(j * _CH, _CH)],
                    sem,
                )
                for d in range(_D)
            ]
            descs.append(
                pltpu.async_copy(
                    cnt.at[rep_v.at[j]], cnt_v.at[pl.ds(j * _CH, _CH)], sem
                )
            )
            for desc in descs:
                desc.wait()
        for g in range(_PER_TILE // _D):
            gs = pl.ds(g * _D, _D)
            recip = 1.0 / cnt_v[gs]
            for d in range(_D):
                val_v[d, gs] = val_v[d, gs] * recip
        for j in range(_NCH):
            descs = [
                pltpu.async_copy(
                    val_v.at[d, pl.ds(j * _CH, _CH)],
                    outw.at[d].at[idx_v.at[j]],
                    sem,
                )
                for d in range(_D)
            ]
            for desc in descs:
                desc.wait()


def kernel(weight, node, update):
    wt = weight.T
    updt = update.T
    node2d = node.reshape(_B // _CH, _CH)
    outw = jax.new_ref(wt)
    embt = _sc_gather(wt, node2d)
    _sc_update(node2d, updt, outw)
    return embt.T, jax.freeze(outw).T


# speedup vs baseline: 6.2064x; 6.2064x over previous
import functools

import jax
import jax.numpy as jnp
from jax import lax
from jax.experimental import pallas as pl
from jax.experimental.pallas import tpu as pltpu
from jax.experimental.pallas import tpu_sc as plsc

_N = 1000000
_D = 16
_B = 16384
_NS = 16
_PER_TILE = _B // _NS
_CH = 128
_NCH = _PER_TILE // _CH
_ROWS_PER_TILE = _PER_TILE // _CH
_TCC = 8192

_mesh = plsc.VectorSubcoreMesh(core_axis_name="c", subcore_axis_name="s")


def _detile(xt, n):
    grid = -(-n // _TCC)

    def body(x_ref, *o_refs):
        x = x_ref[...]
        for d in range(_D):
            o_refs[d][...] = x[d, :]

    return pl.pallas_call(
        body,
        grid=(grid,),
        in_specs=[pl.BlockSpec((_D, _TCC), lambda i: (0, i))],
        out_specs=[pl.BlockSpec((_TCC,), lambda i: (i,)) for _ in range(_D)],
        out_shape=[jax.ShapeDtypeStruct((n,), jnp.float32) for _ in range(_D)],
    )(xt)


def _retile(parts, n):
    grid = -(-n // _TCC)

    def body(*refs):
        o_ref = refs[-1]
        for d in range(_D):
            o_ref[d, :] = refs[d][...]

    return pl.pallas_call(
        body,
        grid=(grid,),
        in_specs=[pl.BlockSpec((_TCC,), lambda i: (i,)) for _ in range(_D)],
        out_specs=pl.BlockSpec((_D, _TCC), lambda i: (0, i)),
        out_shape=jax.ShapeDtypeStruct((_D, n), jnp.float32),
    )(*parts)


@functools.partial(
    pl.kernel,
    out_type=[jax.ShapeDtypeStruct((_B,), jnp.float32) for _ in range(_D)],
    mesh=_mesh,
    scratch_types=[
        pltpu.VMEM((_NCH // 2, _CH), jnp.int32),
        pltpu.VMEM((_D, _PER_TILE // 2), jnp.float32),
        pltpu.SemaphoreType.DMA,
    ],
    compiler_params=pltpu.CompilerParams(use_tc_tiling_on_sc=False, needs_layout_passes=False),
)
def _sc_gather(node2d, *args):
    stages = args[:_D]
    embs = args[_D:2 * _D]
    idx_v, val_v, sem = args[2 * _D:]
    c = lax.axis_index("c")
    s = lax.axis_index("s")
    w = s * 2 + c
    per_w = _PER_TILE // 2
    nch = _NCH // 2
    base = w * per_w
    rowbase = w * (per_w // _CH)

    pltpu.sync_copy(node2d.at[pl.ds(rowbase, nch)], idx_v)
    for j in range(nch):
        descs = [
            pltpu.async_copy(
                stages[d].at[idx_v.at[j]],
                val_v.at[d, pl.ds(j * _CH, _CH)],
                sem,
            )
            for d in range(_D)
        ]
        for desc in descs:
            desc.wait()
    for d in range(_D):
        pltpu.sync_copy(val_v.at[d], embs[d].at[pl.ds(base, per_w)])


@functools.partial(
    pl.kernel,
    out_type=(),
    mesh=_mesh,
    scratch_types=[
        pltpu.VMEM_SHARED((_N,), jnp.int32),
        pltpu.VMEM_SHARED((_B, _D), jnp.float32),
        pltpu.VMEM_SHARED((_B,), jnp.float32),
        pltpu.VMEM((_NCH, _CH), jnp.int32),
        pltpu.VMEM((_NCH, _CH), jnp.int32),
        pltpu.VMEM((_NCH, _CH), jnp.int32),
        pltpu.VMEM((_PER_TILE, _D), jnp.float32),
        pltpu.VMEM((_D, _PER_TILE), jnp.float32),
        pltpu.VMEM((_PER_TILE,), jnp.float32),
        pltpu.VMEM((_PER_TILE,), jnp.float32),
        pltpu.VMEM((_CH, _D), jnp.float32),
        pltpu.VMEM((_CH,), jnp.float32),
        pltpu.VMEM((_CH,), jnp.float32),
        pltpu.SemaphoreType.DMA,
    ],
    compiler_params=pltpu.CompilerParams(use_tc_tiling_on_sc=False, needs_layout_passes=False),
)
def _sc_update(node2d, *args):
    ustages = args[:_D]
    outw = args[_D:2 * _D]
    (slot_tab, acc, cnt, idx_v, ids_v, rep_v, row_v, fm_v, cnt_v, rcp_v,
     zrows_v, zc_v, oc_v, sem) = args[2 * _D:]
    c = lax.axis_index("c")
    s = lax.axis_index("s")
    base = s * _PER_TILE
    rowbase = s * _ROWS_PER_TILE
    ngrp = _PER_TILE // _D
    lanes = lax.iota(jnp.int32, _D)

    @pl.when(c == 0)
    def _():
        pltpu.sync_copy(node2d.at[pl.ds(rowbase, _NCH)], idx_v)
        for j in range(_NCH):
            for g in range(_CH // _D):
                ids_v[j, pl.ds(g * _D, _D)] = base + j * _CH + g * _D + lanes
        for g in range(_CH // _D):
            zc_v[pl.ds(g * _D, _D)] = jnp.zeros((_D,), jnp.float32)
            oc_v[pl.ds(g * _D, _D)] = jnp.ones((_D,), jnp.float32)
        for i in range(_CH):
            zrows_v[i, :] = jnp.zeros((_D,), jnp.float32)
        for d in range(_D):
            pltpu.sync_copy(ustages[d].at[pl.ds(base, _PER_TILE)], fm_v.at[d])

        def _transp(g, _):
            rows = g * _D + lanes
            for d in range(_D):
                v = plsc.load_gather(fm_v, [jnp.full((_D,), d, jnp.int32),
                                            rows])
                plsc.store_scatter(row_v, [rows, jnp.full((_D,), d,
                                                          jnp.int32)], v)
            return 0

        lax.fori_loop(0, ngrp, _transp, 0)
        for j in range(_NCH):
            pltpu.sync_copy(zrows_v, acc.at[pl.ds(base + j * _CH, _CH)])
            pltpu.sync_copy(zc_v, cnt.at[pl.ds(base + j * _CH, _CH)])
        for j in range(_NCH):
            pltpu.sync_copy(ids_v.at[j], slot_tab.at[idx_v.at[j]])

    plsc.subcore_barrier()

    @pl.when(c == 0)
    def _():
        for j in range(_NCH):
            pltpu.sync_copy(slot_tab.at[idx_v.at[j]], rep_v.at[j])
        for j in range(_NCH):
            pltpu.sync_copy(row_v.at[pl.ds(j * _CH, _CH)],
                            acc.at[rep_v.at[j]], add=True)
            pltpu.sync_copy(oc_v, cnt.at[rep_v.at[j]], add=True)

    plsc.subcore_barrier()

    @pl.when(c == 0)
    def _():
        for j in range(_NCH):
            pltpu.sync_copy(acc.at[rep_v.at[j]],
                            row_v.at[pl.ds(j * _CH, _CH)])
            pltpu.sync_copy(cnt.at[rep_v.at[j]],
                            cnt_v.at[pl.ds(j * _CH, _CH)])

        def _recip(g, _):
            gs = pl.ds(g * _D, _D)
            rcp_v[gs] = 1.0 / cnt_v[gs]
            return 0

        lax.fori_loop(0, ngrp, _recip, 0)

        def _transp(g, _):
            rows = g * _D + lanes
            gs = pl.ds(g * _D, _D)
            for d in range(_D):
                v = plsc.load_gather(row_v, [rows, jnp.full((_D,), d,
                                                            jnp.int32)])
                fm_v[d, gs] = v * rcp_v[gs]
            return 0

        lax.fori_loop(0, ngrp, _transp, 0)
        for j in range(_NCH):
            descs = [
                pltpu.async_copy(
                    fm_v.at[d, pl.ds(j * _CH, _CH)],
                    outw[d].at[idx_v.at[j]],
                    sem,
                )
                for d in range(_D)
            ]
            for desc in descs:
                desc.wait()


def kernel(weight, node, update):
    wt = weight.T
    updt = update.T
    node2d = node.reshape(_B // _CH, _CH)
    stages = _detile(wt, _N)
    ustages = _detile(updt, _B)
    outw = [jax.new_ref(st) for st in stages]
    embs = _sc_gather(node2d, *stages)
    _sc_update(node2d, *ustages, *outw)
    embt = _retile(embs, _B)
    new_wt = _retile([jax.freeze(r) for r in outw], _N)
    return embt.T, new_wt.T


# speedup vs baseline: 6.4142x; 1.0335x over previous
import functools

import jax
import jax.numpy as jnp
from jax import lax
from jax.experimental import pallas as pl
from jax.experimental.pallas import tpu as pltpu
from jax.experimental.pallas import tpu_sc as plsc

_N = 1000000
_D = 16
_B = 16384
_NS = 16
_PER_TILE = _B // _NS
_CH = 128
_NCH = _PER_TILE // _CH
_ROWS_PER_TILE = _PER_TILE // _CH
_TCC = 8192

_mesh = plsc.VectorSubcoreMesh(core_axis_name="c", subcore_axis_name="s")


def _detile(xt, n):
    grid = -(-n // _TCC)

    def body(x_ref, *o_refs):
        x = x_ref[...]
        for d in range(_D):
            o_refs[d][...] = x[d, :]

    return pl.pallas_call(
        body,
        grid=(grid,),
        in_specs=[pl.BlockSpec((_D, _TCC), lambda i: (0, i))],
        out_specs=[pl.BlockSpec((_TCC,), lambda i: (i,)) for _ in range(_D)],
        out_shape=[jax.ShapeDtypeStruct((n,), jnp.float32) for _ in range(_D)],
    )(xt)


def _retile(parts, n):
    grid = -(-n // _TCC)

    def body(*refs):
        o_ref = refs[-1]
        for d in range(_D):
            o_ref[d, :] = refs[d][...]

    return pl.pallas_call(
        body,
        grid=(grid,),
        in_specs=[pl.BlockSpec((_TCC,), lambda i: (i,)) for _ in range(_D)],
        out_specs=pl.BlockSpec((_D, _TCC), lambda i: (0, i)),
        out_shape=jax.ShapeDtypeStruct((_D, n), jnp.float32),
    )(*parts)


@functools.partial(
    pl.kernel,
    out_type=[jax.ShapeDtypeStruct((_B,), jnp.float32) for _ in range(_D)],
    mesh=_mesh,
    scratch_types=[
        pltpu.VMEM((_NCH // 2, _CH), jnp.int32),
        pltpu.VMEM((_D, _PER_TILE // 2), jnp.float32),
        pltpu.SemaphoreType.DMA,
    ],
    compiler_params=pltpu.CompilerParams(use_tc_tiling_on_sc=False, needs_layout_passes=False),
)
def _sc_gather(node2d, *args):
    stages = args[:_D]
    embs = args[_D:2 * _D]
    idx_v, val_v, sem = args[2 * _D:]
    c = lax.axis_index("c")
    s = lax.axis_index("s")
    w = s * 2 + c
    per_w = _PER_TILE // 2
    nch = _NCH // 2
    base = w * per_w
    rowbase = w * (per_w // _CH)

    pltpu.sync_copy(node2d.at[pl.ds(rowbase, nch)], idx_v)
    for j in range(nch):
        descs = [
            pltpu.async_copy(
                stages[d].at[idx_v.at[j]],
                val_v.at[d, pl.ds(j * _CH, _CH)],
                sem,
            )
            for d in range(_D)
        ]
        for desc in descs:
            desc.wait()
    for d in range(_D):
        pltpu.sync_copy(val_v.at[d], embs[d].at[pl.ds(base, per_w)])


@functools.partial(
    pl.kernel,
    out_type=(),
    mesh=_mesh,
    scratch_types=[
        pltpu.VMEM_SHARED((_N,), jnp.int32),
        pltpu.VMEM_SHARED((_D // 2, _B), jnp.float32),
        pltpu.VMEM_SHARED((_B,), jnp.float32),
        pltpu.VMEM((_NCH, _CH), jnp.int32),
        pltpu.VMEM((_NCH, _CH), jnp.int32),
        pltpu.VMEM((_NCH, _CH), jnp.int32),
        pltpu.VMEM((_D // 2, _PER_TILE), jnp.float32),
        pltpu.VMEM((_D // 2, _PER_TILE), jnp.float32),
        pltpu.VMEM((_PER_TILE,), jnp.float32),
        pltpu.VMEM((_PER_TILE,), jnp.float32),
        pltpu.VMEM((_PER_TILE,), jnp.float32),
        pltpu.VMEM((_CH,), jnp.float32),
        pltpu.SemaphoreType.DMA,
    ],
    compiler_params=pltpu.CompilerParams(use_tc_tiling_on_sc=False, needs_layout_passes=False),
)
def _sc_update(node2d, *args):
    ustages = args[:_D]
    outw = args[_D:2 * _D]
    (slot_tab, acc, cnt, idx_v, ids_v, rep_v, upd_v, val_v, cnt_v, rcp_v,
     zrow_v, oc_v, sem) = args[2 * _D:]
    c = lax.axis_index("c")
    s = lax.axis_index("s")
    base = s * _PER_TILE
    rowbase = s * _ROWS_PER_TILE
    ngrp = _PER_TILE // _D
    nf = _D // 2
    lanes = lax.iota(jnp.int32, _D)

    def phase_a(fb):
        pltpu.sync_copy(node2d.at[pl.ds(rowbase, _NCH)], idx_v)
        for j in range(_NCH):
            for g in range(_CH // _D):
                ids_v[j, pl.ds(g * _D, _D)] = base + j * _CH + g * _D + lanes
        for g in range(_CH // _D):
            oc_v[pl.ds(g * _D, _D)] = jnp.ones((_D,), jnp.float32)
        for g in range(ngrp):
            zrow_v[pl.ds(g * _D, _D)] = jnp.zeros((_D,), jnp.float32)
        for d in range(nf):
            pltpu.sync_copy(ustages[fb + d].at[pl.ds(base, _PER_TILE)],
                            upd_v.at[d])
        for d in range(nf):
            pltpu.sync_copy(zrow_v, acc.at[d, pl.ds(base, _PER_TILE)])
        pltpu.sync_copy(zrow_v, cnt.at[pl.ds(base, _PER_TILE)])
        for j in range(_NCH):
            pltpu.sync_copy(ids_v.at[j], slot_tab.at[idx_v.at[j]])

    def phase_b(fb):
        for j in range(_NCH):
            pltpu.sync_copy(slot_tab.at[idx_v.at[j]], rep_v.at[j])
        for j in range(_NCH):
            descs = [
                pltpu.async_copy(upd_v.at[d, pl.ds(j * _CH, _CH)],
                                 acc.at[d].at[rep_v.at[j]], sem, add=True)
                for d in range(nf)
            ]
            descs.append(pltpu.async_copy(oc_v, cnt.at[rep_v.at[j]], sem,
                                          add=True))
            for desc in descs:
                desc.wait()

    def phase_c(fb):
        for j in range(_NCH):
            descs = [
                pltpu.async_copy(acc.at[d].at[rep_v.at[j]],
                                 val_v.at[d, pl.ds(j * _CH, _CH)], sem)
                for d in range(nf)
            ]
            descs.append(pltpu.async_copy(cnt.at[rep_v.at[j]],
                                          cnt_v.at[pl.ds(j * _CH, _CH)], sem))
            for desc in descs:
                desc.wait()

        def _recip(g, _):
            gs = pl.ds(g * _D, _D)
            rcp_v[gs] = 1.0 / cnt_v[gs]
            return 0

        lax.fori_loop(0, ngrp, _recip, 0)

        def _scale(g, _):
            gs = pl.ds(g * _D, _D)
            r = rcp_v[gs]
            for d in range(nf):
                val_v[d, gs] = val_v[d, gs] * r
            return 0

        lax.fori_loop(0, ngrp, _scale, 0)
        for j in range(_NCH):
            descs = [
                pltpu.async_copy(val_v.at[d, pl.ds(j * _CH, _CH)],
                                 outw[fb + d].at[idx_v.at[j]], sem)
                for d in range(nf)
            ]
            for desc in descs:
                desc.wait()

    @pl.when(c == 0)
    def _():
        phase_a(0)

    @pl.when(c == 1)
    def _():
        phase_a(nf)

    plsc.subcore_barrier()

    @pl.when(c == 0)
    def _():
        phase_b(0)

    @pl.when(c == 1)
    def _():
        phase_b(nf)

    plsc.subcore_barrier()

    @pl.when(c == 0)
    def _():
        phase_c(0)

    @pl.when(c == 1)
    def _():
        phase_c(nf)


def kernel(weight, node, update):
    wt = weight.T
    updt = update.T
    node2d = node.reshape(_B // _CH, _CH)
    stages = _detile(wt, _N)
    ustages = _detile(updt, _B)
    outw = [jax.new_ref(st) for st in stages]
    embs = _sc_gather(node2d, *stages)
    _sc_update(node2d, *ustages, *outw)
    embt = _retile(embs, _B)
    new_wt = _retile([jax.freeze(r) for r in outw], _N)
    return embt.T, new_wt.T
